# Initial kernel scaffold; baseline (speedup 1.0000x reference)
import jax
import jax.numpy as jnp
from jax.experimental import pallas as pl


def kernel(s_emb, q_emb, alpha_msg, alpha_proto, Wq, bq, Wk, bk, Wv, bv):
    raise NotImplementedError("write your pallas kernel here")


# speedup vs baseline: 13.1758x; 13.1758x over previous
import functools

import jax
import jax.numpy as jnp
from jax.experimental import pallas as pl
from jax.experimental.pallas import tpu as pltpu

_TOP_K = 32
_BR = 256
_BQ = 2048


def _sort_key(x_f32):
    b = jax.lax.bitcast_convert_type(x_f32, jnp.int32)
    return b ^ jax.lax.shift_right_arithmetic(b, 31).astype(jnp.int32) & jnp.int32(
        0x7FFFFFFF
    )


def _norm_body(s_ref, o_ref):
    x = s_ref[...]
    n = jnp.sqrt(jnp.sum(x * x, axis=1, keepdims=True))
    o_ref[...] = x / n


def _kth_threshold(key, k):
    lo = jnp.min(key, axis=1, keepdims=True)
    hi = jnp.max(key, axis=1, keepdims=True)

    def body(_, lh):
        lo, hi = lh
        mid = lo + jax.lax.shift_right_arithmetic(hi - lo + 1, 1)
        cnt = jnp.sum((key >= mid).astype(jnp.int32), axis=1, keepdims=True)
        ge = cnt >= k
        return jnp.where(ge, mid, lo), jnp.where(ge, hi, mid - 1)

    lo, hi = jax.lax.fori_loop(0, 31, body, (lo, hi))
    return lo


def _agg_body(alpha_ref, s_ref, xs_ref, o_ref, *, br, k):
    i = pl.program_id(0)
    s_all = s_ref[...]
    xs_all = xs_ref[...]
    xs_blk = xs_ref[pl.ds(i * br, br), :]
    s_blk = s_ref[pl.ds(i * br, br), :]

    sim = jax.lax.dot_general(
        xs_blk, xs_all, (((1,), (1,)), ((), ())),
        preferred_element_type=jnp.float32,
    )
    key = _sort_key(sim)
    thr = _kth_threshold(key, k)
    mask = (key >= thr).astype(jnp.float32)
    agg = jax.lax.dot_general(
        mask, s_all, (((1,), (0,)), ((), ())),
        preferred_element_type=jnp.float32,
    )
    o_ref[...] = s_blk + alpha_ref[0, 0] * agg


def _attn_body(alpha_ref, s_ref, q_ref, wq_ref, bq_ref, wk_ref, bk_ref,
               wv_ref, bv_ref, o_ref):
    s_all = s_ref[...]
    n = s_all.shape[0]
    half = n // 2
    pos = jnp.mean(s_all[half:, :], axis=0, keepdims=True)
    neg = jnp.mean(s_all[:half, :], axis=0, keepdims=True)

    q = q_ref[...]
    dn = (((1,), (1,)), ((), ()))
    Q = jax.lax.dot_general(q, wq_ref[...], dn,
                            preferred_element_type=jnp.float32) + bq_ref[...]
    k_pos = jax.lax.dot_general(pos, wk_ref[...], dn,
                                preferred_element_type=jnp.float32) + bk_ref[...]
    k_neg = jax.lax.dot_general(neg, wk_ref[...], dn,
                                preferred_element_type=jnp.float32) + bk_ref[...]
    v_pos = jax.lax.dot_general(pos, wv_ref[...], dn,
                                preferred_element_type=jnp.float32) + bv_ref[...]
    v_neg = jax.lax.dot_general(neg, wv_ref[...], dn,
                                preferred_element_type=jnp.float32) + bv_ref[...]

    scale = jnp.float32(q.shape[1]) ** 0.5
    l_pos = jnp.sum(Q * k_pos, axis=1, keepdims=True) / scale
    l_neg = jnp.sum(Q * k_neg, axis=1, keepdims=True) / scale
    m = jnp.maximum(l_pos, l_neg)
    e_pos = jnp.exp(l_pos - m)
    e_neg = jnp.exp(l_neg - m)
    denom = e_pos + e_neg
    ctx = (e_pos / denom) * v_pos + (e_neg / denom) * v_neg
    o_ref[...] = q + alpha_ref[0, 0] * ctx


def kernel(s_emb, q_emb, alpha_msg, alpha_proto, Wq, bq, Wk, bk, Wv, bv):
    n, d = s_emb.shape
    nq = q_emb.shape[0]
    br = _BR if n % _BR == 0 else n
    bq_blk = _BQ if nq % _BQ == 0 else nq

    xs = pl.pallas_call(
        _norm_body,
        out_shape=jax.ShapeDtypeStruct((n, d), jnp.float32),
    )(s_emb)

    am = jnp.reshape(alpha_msg, (1, 1)).astype(jnp.float32)
    ap = jnp.reshape(alpha_proto, (1, 1)).astype(jnp.float32)

    full = lambda r, c: pl.BlockSpec((r, c), lambda i: (0, 0))
    enhanced_s = pl.pallas_call(
        functools.partial(_agg_body, br=br, k=_TOP_K),
        grid=(n // br,),
        in_specs=[
            full(1, 1),
            full(n, d),
            full(n, d),
        ],
        out_specs=pl.BlockSpec((br, d), lambda i: (i, 0)),
        out_shape=jax.ShapeDtypeStruct((n, d), jnp.float32),
        compiler_params=pltpu.CompilerParams(
            dimension_semantics=("arbitrary",),
        ),
    )(am, s_emb, xs)

    enhanced_q = pl.pallas_call(
        _attn_body,
        grid=(nq // bq_blk,),
        in_specs=[
            full(1, 1),
            full(n, d),
            pl.BlockSpec((bq_blk, d), lambda i: (i, 0)),
            full(d, d),
            full(1, d),
            full(d, d),
            full(1, d),
            full(d, d),
            full(1, d),
        ],
        out_specs=pl.BlockSpec((bq_blk, d), lambda i: (i, 0)),
        out_shape=jax.ShapeDtypeStruct((nq, d), jnp.float32),
        compiler_params=pltpu.CompilerParams(
            dimension_semantics=("arbitrary",),
        ),
    )(ap, s_emb, q_emb, Wq, jnp.reshape(bq, (1, d)), Wk, jnp.reshape(bk, (1, d)),
      Wv, jnp.reshape(bv, (1, d)))

    return (enhanced_s, enhanced_q)
